# baseline (speedup 1.0000x reference)
import functools

import jax
import jax.numpy as jnp
from jax import lax
from jax.experimental import pallas as pl
from jax.experimental.pallas import tpu as pltpu
from jax.experimental.pallas import tpu_sc as plsc

BUCKET = 64
N_HASHES = 8
SELF_VAL = -5e4
CPG = 4

SC_NC = 2
SC_NS = 16
SC_NW = SC_NC * SC_NS
GCHUNK = 128


def _sc_gather_body(nt, iters, idx_hbm, *rest):
    tabs = rest[:nt]
    outs = rest[nt:2 * nt]
    idx_v = rest[2 * nt]
    rows = rest[2 * nt + 1:3 * nt + 1]
    sems = rest[3 * nt + 1:]
    wid = lax.axis_index("s") * SC_NC + lax.axis_index("c")
    base = wid * (iters * GCHUNK)

    def step(i, _):
        start = pl.multiple_of(base + i * GCHUNK, GCHUNK)
        pltpu.sync_copy(idx_hbm.at[pl.ds(start, GCHUNK)], idx_v)
        cps = [pltpu.async_copy(tabs[j].at[idx_v], rows[j], sems[j])
               for j in range(nt)]
        for j in range(nt):
            cps[j].wait()
            pltpu.sync_copy(rows[j], outs[j].at[pl.ds(start, GCHUNK)])
        return _

    lax.fori_loop(0, iters, step, None)


def _sc_gather_rows(idx_flat, *tables):
    n = idx_flat.shape[0]
    nt = len(tables)
    d = tables[0].shape[1]
    iters = n // (SC_NW * GCHUNK)
    mesh = plsc.VectorSubcoreMesh(core_axis_name="c", subcore_axis_name="s")
    f = pl.kernel(
        functools.partial(_sc_gather_body, nt, iters),
        out_type=[jax.ShapeDtypeStruct((n, d), jnp.float32)] * nt,
        mesh=mesh,
        scratch_types=(
            [pltpu.VMEM((GCHUNK,), jnp.int32)]
            + [pltpu.VMEM((GCHUNK, d), jnp.float32)] * nt
            + [pltpu.SemaphoreType.DMA] * nt
        ),
    )
    return f(idx_flat, *tables)


def _attn_body(cq, pq, cv, pv, ct, pt, qtc, bias, so_ref, lse_ref, *, dim):
    scale = dim ** -0.5
    nq = CPG * BUCKET
    q = cq[0]
    kraw = jnp.concatenate([pq[0], q], axis=0)
    norm = jnp.sum(kraw * kraw, axis=-1, keepdims=True)
    kn = kraw * (scale / jnp.maximum(jnp.sqrt(norm), 1e-12))
    vall = jnp.concatenate([pv[0], cv[0]], axis=0)
    dots = jax.lax.dot_general(
        q, kn, (((1,), (1,)), ((), ()))) + bias[...]
    qt = qtc[0]
    kt01 = jnp.concatenate([pt[0, 0, :, :], ct[0, 0:1, 0, :]],
                           axis=1)
    left = jnp.where(qt == kt01, -1e9, dots[:, :2 * BUCKET])
    dots = jnp.concatenate([left, dots[:, 2 * BUCKET:]], axis=1)
    ex = jnp.exp(dots)
    s = jnp.sum(ex, axis=-1, keepdims=True)
    bo = jax.lax.dot_general(ex, vall, (((1,), (0,)), ((), ())))
    so_ref[0] = bo / s
    lse_ref[0] = jnp.log(s)


def _band_bias():
    import numpy as np
    nq = CPG * BUCKET
    r = np.arange(nq)[:, None]
    c = np.arange(nq + BUCKET)[None, :]
    band = (c // BUCKET == r // BUCKET) | (c // BUCKET == r // BUCKET + 1)
    diag = c == r + BUCKET
    return jnp.asarray(
        np.where(band & ~diag, 0.0, -1e9).astype(np.float32))


def _attention(sqk, sv, st, n_chunks):
    b, n, d = sqk.shape
    nj = n_chunks // CPG
    st4 = st.reshape(b, n_chunks, 1, BUCKET)

    def im_cur(bi, j):
        return (bi, j, 0)

    def im_prev(bi, j):
        return (bi, (j * CPG + n_chunks - 1) % n_chunks, 0)

    def im_cur4(bi, j):
        return (bi, j, 0, 0)

    def im_prev4(bi, j):
        return (bi, (j * CPG + n_chunks - 1) % n_chunks, 0, 0)

    so, lse = pl.pallas_call(
        functools.partial(_attn_body, dim=d),
        grid=(b, nj),
        in_specs=[
            pl.BlockSpec((1, CPG * BUCKET, d), im_cur),
            pl.BlockSpec((1, BUCKET, d), im_prev),
            pl.BlockSpec((1, CPG * BUCKET, d), im_cur),
            pl.BlockSpec((1, BUCKET, d), im_prev),
            pl.BlockSpec((1, CPG, 1, BUCKET), im_cur4),
            pl.BlockSpec((1, 1, 1, BUCKET), im_prev4),
            pl.BlockSpec((1, CPG * BUCKET, 1), im_cur),
            pl.BlockSpec((CPG * BUCKET, (CPG + 1) * BUCKET),
                         lambda bi, j: (0, 0)),
        ],
        out_specs=[
            pl.BlockSpec((1, CPG * BUCKET, d), im_cur),
            pl.BlockSpec((1, CPG * BUCKET, 1), im_cur),
        ],
        out_shape=[
            jax.ShapeDtypeStruct((b, n, d), jnp.float32),
            jax.ShapeDtypeStruct((b, n, 1), jnp.float32),
        ],
    )(sqk, sqk, sv, sv, st4, st4, st.reshape(b, n, 1), _band_bias())
    return so, lse[..., 0]


def kernel(qk, v, rot):
    b, t, d = qk.shape
    n_buckets = t // BUCKET
    rr = jnp.broadcast_to(rot, (b,) + rot.shape[1:])
    rotated = jnp.einsum('btf,bfhi->bhti', qk, rr)
    rotated = jnp.concatenate([rotated, -rotated], axis=-1)
    buckets = jnp.argmax(rotated, axis=-1)
    offsets = (jnp.arange(N_HASHES) * n_buckets).reshape(1, -1, 1)
    buckets = (buckets + offsets).reshape(b, -1)
    ticker = jnp.broadcast_to(jnp.arange(N_HASHES * t)[None, :], buckets.shape)
    buckets_and_t = t * buckets + ticker % t
    sticker = jnp.argsort(buckets_and_t, axis=-1)
    undo = jnp.argsort(sticker, axis=-1)
    st = (sticker % t).astype(jnp.int32)
    gidx = (st + (jnp.arange(b, dtype=jnp.int32) * t)[:, None]).reshape(-1)
    sqk_f, sv_f = _sc_gather_rows(gidx, qk.reshape(b * t, d), v.reshape(b * t, d))
    sqk = sqk_f.reshape(b, N_HASHES * t, d)
    sv = sv_f.reshape(b, N_HASHES * t, d)
    n_chunks = N_HASHES * n_buckets
    so, slse = _attention(sqk, sv, st, n_chunks)
    n = N_HASHES * t
    uidx = (undo.astype(jnp.int32)
            + (jnp.arange(b, dtype=jnp.int32) * n)[:, None]).reshape(-1)
    (o_f,) = _sc_gather_rows(uidx, so.reshape(b * n, d))
    o = o_f.reshape(b, n, d)
    logits = jnp.take_along_axis(slse, undo, axis=1)
    o = o.reshape(b, N_HASHES, t, d)
    logits = logits.reshape(b, N_HASHES, t, 1)
    probs = jnp.exp(
        logits - jax.scipy.special.logsumexp(logits, axis=1, keepdims=True))
    return jnp.sum(o * probs, axis=1)


# speedup vs baseline: 1.2413x; 1.2413x over previous
import functools

import jax
import jax.numpy as jnp
from jax import lax
from jax.experimental import pallas as pl
from jax.experimental.pallas import tpu as pltpu
from jax.experimental.pallas import tpu_sc as plsc

BUCKET = 64
N_HASHES = 8
SELF_VAL = -5e4
CPG = 8

SC_NC = 2
SC_NS = 16
SC_NW = SC_NC * SC_NS
GCHUNK = 128


def _sc_gather_body(nt, iters, idx_hbm, *rest):
    tabs = rest[:nt]
    outs = rest[nt:2 * nt]
    idx_v = rest[2 * nt]
    rows = rest[2 * nt + 1:3 * nt + 1]
    sems = rest[3 * nt + 1:]
    wid = lax.axis_index("s") * SC_NC + lax.axis_index("c")
    base = wid * (iters * GCHUNK)

    def step(i, _):
        start = pl.multiple_of(base + i * GCHUNK, GCHUNK)
        pltpu.sync_copy(idx_hbm.at[pl.ds(start, GCHUNK)], idx_v)
        cps = [pltpu.async_copy(tabs[j].at[idx_v], rows[j], sems[j])
               for j in range(nt)]
        for j in range(nt):
            cps[j].wait()
            pltpu.sync_copy(rows[j], outs[j].at[pl.ds(start, GCHUNK)])
        return _

    lax.fori_loop(0, iters, step, None)


def _sc_gather_rows(idx_flat, *tables):
    n = idx_flat.shape[0]
    nt = len(tables)
    d = tables[0].shape[1]
    iters = n // (SC_NW * GCHUNK)
    mesh = plsc.VectorSubcoreMesh(core_axis_name="c", subcore_axis_name="s")
    f = pl.kernel(
        functools.partial(_sc_gather_body, nt, iters),
        out_type=[jax.ShapeDtypeStruct((n, d), jnp.float32)] * nt,
        mesh=mesh,
        scratch_types=(
            [pltpu.VMEM((GCHUNK,), jnp.int32)]
            + [pltpu.VMEM((GCHUNK, d), jnp.float32)] * nt
            + [pltpu.SemaphoreType.DMA] * nt
        ),
    )
    return f(idx_flat, *tables)


def _attn_body(cq, pq, cv, pv, ct, pt, qtc, bias, so_ref, lse_ref, *, dim):
    scale = dim ** -0.5
    nq = CPG * BUCKET
    q = cq[0]
    kraw = jnp.concatenate([pq[0], q], axis=0)
    norm = jnp.sum(kraw * kraw, axis=-1, keepdims=True)
    kn = kraw * (scale / jnp.maximum(jnp.sqrt(norm), 1e-12))
    vall = jnp.concatenate([pv[0], cv[0]], axis=0)
    dots = jax.lax.dot_general(
        q, kn, (((1,), (1,)), ((), ()))) + bias[...]
    qt = qtc[0]
    kt01 = jnp.concatenate([pt[0, 0, :, :], ct[0, 0:1, 0, :]],
                           axis=1)
    left = jnp.where(qt == kt01, -1e9, dots[:, :2 * BUCKET])
    dots = jnp.concatenate([left, dots[:, 2 * BUCKET:]], axis=1)
    ex = jnp.exp(dots)
    s = jnp.sum(ex, axis=-1, keepdims=True)
    bo = jax.lax.dot_general(ex, vall, (((1,), (0,)), ((), ())))
    so_ref[0] = bo / s
    lse_ref[0] = jnp.log(s)


def _band_bias():
    import numpy as np
    nq = CPG * BUCKET
    r = np.arange(nq)[:, None]
    c = np.arange(nq + BUCKET)[None, :]
    band = (c // BUCKET == r // BUCKET) | (c // BUCKET == r // BUCKET + 1)
    diag = c == r + BUCKET
    return jnp.asarray(
        np.where(band & ~diag, 0.0, -1e9).astype(np.float32))


def _attention(sqk, sv, st, n_chunks):
    b, n, d = sqk.shape
    nj = n_chunks // CPG
    st4 = st.reshape(b, n_chunks, 1, BUCKET)

    def im_cur(bi, j):
        return (bi, j, 0)

    def im_prev(bi, j):
        return (bi, (j * CPG + n_chunks - 1) % n_chunks, 0)

    def im_cur4(bi, j):
        return (bi, j, 0, 0)

    def im_prev4(bi, j):
        return (bi, (j * CPG + n_chunks - 1) % n_chunks, 0, 0)

    so, lse = pl.pallas_call(
        functools.partial(_attn_body, dim=d),
        grid=(b, nj),
        in_specs=[
            pl.BlockSpec((1, CPG * BUCKET, d), im_cur),
            pl.BlockSpec((1, BUCKET, d), im_prev),
            pl.BlockSpec((1, CPG * BUCKET, d), im_cur),
            pl.BlockSpec((1, BUCKET, d), im_prev),
            pl.BlockSpec((1, CPG, 1, BUCKET), im_cur4),
            pl.BlockSpec((1, 1, 1, BUCKET), im_prev4),
            pl.BlockSpec((1, CPG * BUCKET, 1), im_cur),
            pl.BlockSpec((CPG * BUCKET, (CPG + 1) * BUCKET),
                         lambda bi, j: (0, 0)),
        ],
        out_specs=[
            pl.BlockSpec((1, CPG * BUCKET, d), im_cur),
            pl.BlockSpec((1, CPG * BUCKET, 1), im_cur),
        ],
        out_shape=[
            jax.ShapeDtypeStruct((b, n, d), jnp.float32),
            jax.ShapeDtypeStruct((b, n, 1), jnp.float32),
        ],
    )(sqk, sqk, sv, sv, st4, st4, st.reshape(b, n, 1), _band_bias())
    return so, lse[..., 0]


def kernel(qk, v, rot):
    b, t, d = qk.shape
    n_buckets = t // BUCKET
    rr = jnp.broadcast_to(rot, (b,) + rot.shape[1:])
    rotated = jnp.einsum('btf,bfhi->bhti', qk, rr)
    rotated = jnp.concatenate([rotated, -rotated], axis=-1)
    buckets = jnp.argmax(rotated, axis=-1)
    offsets = (jnp.arange(N_HASHES) * n_buckets).reshape(1, -1, 1)
    buckets = (buckets + offsets).reshape(b, -1)
    ticker = jnp.broadcast_to(jnp.arange(N_HASHES * t)[None, :], buckets.shape)
    buckets_and_t = t * buckets + ticker % t
    sticker = jnp.argsort(buckets_and_t, axis=-1)
    undo = jnp.argsort(sticker, axis=-1)
    st = (sticker % t).astype(jnp.int32)
    gidx = (st + (jnp.arange(b, dtype=jnp.int32) * t)[:, None]).reshape(-1)
    sqk_f, sv_f = _sc_gather_rows(gidx, qk.reshape(b * t, d), v.reshape(b * t, d))
    sqk = sqk_f.reshape(b, N_HASHES * t, d)
    sv = sv_f.reshape(b, N_HASHES * t, d)
    n_chunks = N_HASHES * n_buckets
    so, slse = _attention(sqk, sv, st, n_chunks)
    n = N_HASHES * t
    uidx = (undo.astype(jnp.int32)
            + (jnp.arange(b, dtype=jnp.int32) * n)[:, None]).reshape(-1)
    (o_f,) = _sc_gather_rows(uidx, so.reshape(b * n, d))
    o = o_f.reshape(b, n, d)
    logits = jnp.take_along_axis(slse, undo, axis=1)
    o = o.reshape(b, N_HASHES, t, d)
    logits = logits.reshape(b, N_HASHES, t, 1)
    probs = jnp.exp(
        logits - jax.scipy.special.logsumexp(logits, axis=1, keepdims=True))
    return jnp.sum(o * probs, axis=1)


# speedup vs baseline: 1.3361x; 1.0764x over previous
import functools

import jax
import jax.numpy as jnp
from jax import lax
from jax.experimental import pallas as pl
from jax.experimental.pallas import tpu as pltpu
from jax.experimental.pallas import tpu_sc as plsc

BUCKET = 64
N_HASHES = 8
SELF_VAL = -5e4
CPG = 8

SC_NC = 2
SC_NS = 16
SC_NW = SC_NC * SC_NS
GCHUNK = 128


def _sc_gather_body(nt, iters, idx_hbm, *rest):
    tabs = rest[:nt]
    outs = rest[nt:2 * nt]
    idx_v = rest[2 * nt:2 * nt + 2]
    rows = rest[2 * nt + 2:4 * nt + 2]
    gsems = rest[4 * nt + 2:6 * nt + 2]
    wsems = rest[6 * nt + 2:8 * nt + 2]
    wid = lax.axis_index("s") * SC_NC + lax.axis_index("c")
    base = wid * (iters * GCHUNK)

    def cs(i):
        return pl.multiple_of(base + i * GCHUNK, GCHUNK)

    def load_idx(i, s):
        pltpu.sync_copy(idx_hbm.at[pl.ds(cs(i), GCHUNK)], idx_v[s])

    def g_start(s):
        for j in range(nt):
            pltpu.async_copy(tabs[j].at[idx_v[s]], rows[2 * j + s],
                             gsems[2 * j + s])

    def g_wait(s):
        for j in range(nt):
            pltpu.make_async_copy(tabs[j].at[idx_v[s]], rows[2 * j + s],
                                  gsems[2 * j + s]).wait()

    def w_start(i, s):
        for j in range(nt):
            pltpu.async_copy(rows[2 * j + s], outs[j].at[pl.ds(cs(i), GCHUNK)],
                             wsems[2 * j + s])

    def w_wait(i, s):
        for j in range(nt):
            pltpu.make_async_copy(rows[2 * j + s],
                                  outs[j].at[pl.ds(cs(i), GCHUNK)],
                                  wsems[2 * j + s]).wait()

    for s in (0, 1):
        load_idx(s, s)
        g_start(s)

    def body(i2, carry):
        i = i2 * 2
        for s in (0, 1):
            g_wait(s)
            w_start(i + s, s)
        for s in (0, 1):
            w_wait(i + s, s)
            load_idx(i + 2 + s, s)
            g_start(s)
        return carry

    lax.fori_loop(0, iters // 2 - 1, body, None)
    i_last = iters - 2
    for s in (0, 1):
        g_wait(s)
        w_start(i_last + s, s)
    for s in (0, 1):
        w_wait(i_last + s, s)


def _sc_gather_rows(idx_flat, *tables):
    n = idx_flat.shape[0]
    nt = len(tables)
    d = tables[0].shape[1]
    iters = n // (SC_NW * GCHUNK)
    mesh = plsc.VectorSubcoreMesh(core_axis_name="c", subcore_axis_name="s")
    f = pl.kernel(
        functools.partial(_sc_gather_body, nt, iters),
        out_type=[jax.ShapeDtypeStruct((n, d), jnp.float32)] * nt,
        mesh=mesh,
        scratch_types=(
            [pltpu.VMEM((GCHUNK,), jnp.int32)] * 2
            + [pltpu.VMEM((GCHUNK, d), jnp.float32)] * (2 * nt)
            + [pltpu.SemaphoreType.DMA] * (4 * nt)
        ),
    )
    return f(idx_flat, *tables)


def _attn_body(cq, pq, cv, pv, ct, pt, qtc, bias, so_ref, lse_ref, *, dim):
    scale = dim ** -0.5
    nq = CPG * BUCKET
    q = cq[0]
    kraw = jnp.concatenate([pq[0], q], axis=0)
    norm = jnp.sum(kraw * kraw, axis=-1, keepdims=True)
    kn = kraw * (scale / jnp.maximum(jnp.sqrt(norm), 1e-12))
    vall = jnp.concatenate([pv[0], cv[0]], axis=0)
    dots = jax.lax.dot_general(
        q, kn, (((1,), (1,)), ((), ()))) + bias[...]
    qt = qtc[0]
    kt01 = jnp.concatenate([pt[0, 0, :, :], ct[0, 0:1, 0, :]],
                           axis=1)
    left = jnp.where(qt == kt01, -1e9, dots[:, :2 * BUCKET])
    dots = jnp.concatenate([left, dots[:, 2 * BUCKET:]], axis=1)
    ex = jnp.exp(dots)
    s = jnp.sum(ex, axis=-1, keepdims=True)
    bo = jax.lax.dot_general(ex, vall, (((1,), (0,)), ((), ())))
    so_ref[0] = bo / s
    lse_ref[0] = jnp.log(s)


def _band_bias():
    import numpy as np
    nq = CPG * BUCKET
    r = np.arange(nq)[:, None]
    c = np.arange(nq + BUCKET)[None, :]
    band = (c // BUCKET == r // BUCKET) | (c // BUCKET == r // BUCKET + 1)
    diag = c == r + BUCKET
    return jnp.asarray(
        np.where(band & ~diag, 0.0, -1e9).astype(np.float32))


def _attention(sqk, sv, st, n_chunks):
    b, n, d = sqk.shape
    nj = n_chunks // CPG
    st4 = st.reshape(b, n_chunks, 1, BUCKET)

    def im_cur(bi, j):
        return (bi, j, 0)

    def im_prev(bi, j):
        return (bi, (j * CPG + n_chunks - 1) % n_chunks, 0)

    def im_cur4(bi, j):
        return (bi, j, 0, 0)

    def im_prev4(bi, j):
        return (bi, (j * CPG + n_chunks - 1) % n_chunks, 0, 0)

    so, lse = pl.pallas_call(
        functools.partial(_attn_body, dim=d),
        grid=(b, nj),
        in_specs=[
            pl.BlockSpec((1, CPG * BUCKET, d), im_cur),
            pl.BlockSpec((1, BUCKET, d), im_prev),
            pl.BlockSpec((1, CPG * BUCKET, d), im_cur),
            pl.BlockSpec((1, BUCKET, d), im_prev),
            pl.BlockSpec((1, CPG, 1, BUCKET), im_cur4),
            pl.BlockSpec((1, 1, 1, BUCKET), im_prev4),
            pl.BlockSpec((1, CPG * BUCKET, 1), im_cur),
            pl.BlockSpec((CPG * BUCKET, (CPG + 1) * BUCKET),
                         lambda bi, j: (0, 0)),
        ],
        out_specs=[
            pl.BlockSpec((1, CPG * BUCKET, d), im_cur),
            pl.BlockSpec((1, CPG * BUCKET, 1), im_cur),
        ],
        out_shape=[
            jax.ShapeDtypeStruct((b, n, d), jnp.float32),
            jax.ShapeDtypeStruct((b, n, 1), jnp.float32),
        ],
    )(sqk, sqk, sv, sv, st4, st4, st.reshape(b, n, 1), _band_bias())
    return so, lse[..., 0]


def kernel(qk, v, rot):
    b, t, d = qk.shape
    n_buckets = t // BUCKET
    rr = jnp.broadcast_to(rot, (b,) + rot.shape[1:])
    rotated = jnp.einsum('btf,bfhi->bhti', qk, rr)
    rotated = jnp.concatenate([rotated, -rotated], axis=-1)
    buckets = jnp.argmax(rotated, axis=-1)
    offsets = (jnp.arange(N_HASHES) * n_buckets).reshape(1, -1, 1)
    buckets = (buckets + offsets).reshape(b, -1)
    ticker = jnp.broadcast_to(jnp.arange(N_HASHES * t)[None, :], buckets.shape)
    buckets_and_t = t * buckets + ticker % t
    sticker = jnp.argsort(buckets_and_t, axis=-1)
    undo = jnp.argsort(sticker, axis=-1)
    st = (sticker % t).astype(jnp.int32)
    gidx = (st + (jnp.arange(b, dtype=jnp.int32) * t)[:, None]).reshape(-1)
    sqk_f, sv_f = _sc_gather_rows(gidx, qk.reshape(b * t, d), v.reshape(b * t, d))
    sqk = sqk_f.reshape(b, N_HASHES * t, d)
    sv = sv_f.reshape(b, N_HASHES * t, d)
    n_chunks = N_HASHES * n_buckets
    so, slse = _attention(sqk, sv, st, n_chunks)
    n = N_HASHES * t
    uidx = (undo.astype(jnp.int32)
            + (jnp.arange(b, dtype=jnp.int32) * n)[:, None]).reshape(-1)
    (o_f,) = _sc_gather_rows(uidx, so.reshape(b * n, d))
    o = o_f.reshape(b, n, d)
    logits = jnp.take_along_axis(slse, undo, axis=1)
    o = o.reshape(b, N_HASHES, t, d)
    logits = logits.reshape(b, N_HASHES, t, 1)
    probs = jnp.exp(
        logits - jax.scipy.special.logsumexp(logits, axis=1, keepdims=True))
    return jnp.sum(o * probs, axis=1)
